# baseline (speedup 1.0000x reference)
import functools

import jax
import jax.numpy as jnp
from jax import lax
from jax.experimental import pallas as pl
from jax.experimental.pallas import tpu as pltpu
from jax.experimental.pallas import tpu_sc as plsc

B = 16384
NF = 26
VOCAB = 100000
E = 16
ND = 100
NCONT = ND - NF
OUT = NF * E + NCONT
EMB = NF * E

VBLK = 33408
GBLK = VBLK // 8
GPF = 3 * GBLK

NC = 2
NS = 16
NW = NC * NS
BPW = B // NW
R = 128
NCHUNK = BPW // R


def _tc_body(wt_ref, o_ref):
    x = wt_ref[0]
    xt = x.T.reshape(GBLK, 8, 16)
    o_ref[0] = jnp.concatenate([xt[:, k, :] for k in range(8)], axis=1)


def _tc_relayout(wt):
    return pl.pallas_call(
        _tc_body,
        grid=(NF, (VOCAB + VBLK - 1) // VBLK),
        in_specs=[pl.BlockSpec((1, E, VBLK), lambda f, j: (f, 0, j))],
        out_specs=pl.BlockSpec((1, GBLK, 128), lambda f, j: (f, j, 0)),
        out_shape=jax.ShapeDtypeStruct((NF, GPF, 128), jnp.float32),
    )(wt)


@functools.partial(
    pl.kernel,
    mesh=plsc.VectorSubcoreMesh(core_axis_name="c", subcore_axis_name="s"),
    out_type=jax.ShapeDtypeStruct((B, OUT), jnp.float32),
    compiler_params=pltpu.CompilerParams(
        use_tc_tiling_on_sc=False, needs_layout_passes=False
    ),
    scratch_types=[
        pltpu.VMEM((R // 2 * ND,), jnp.int32),
        pltpu.VMEM((NF * R,), jnp.int32),
        pltpu.VMEM((NF * R,), jnp.int32),
        pltpu.VMEM((R, 128), jnp.float32),
        pltpu.VMEM((R, 128), jnp.float32),
        pltpu.VMEM((R, OUT), jnp.float32),
        pltpu.SemaphoreType.DMA,
        pltpu.SemaphoreType.DMA,
    ],
)
def _sc_embed(x_hbm, w_hbm, out_hbm, x_v, idx_v, off_v, g0_v, g1_v, out_v,
              sem_in, sem_g):
    wid = lax.axis_index("s") * NC + lax.axis_index("c")
    iota = lax.iota(jnp.int32, 16)
    colstep = iota * ND

    H = R // 2
    gbufs = [g0_v, g1_v]

    def chunk_body(c, _):
        base = wid * BPW + c * R

        for h in range(2):
            pltpu.async_copy(
                x_hbm.at[pl.ds((base + h * H) * ND, H * ND)], x_v, sem_in
            ).wait()

            for f in range(NF):
                for i in range(H // 16):
                    codes = plsc.load_gather(x_v, [colstep + (i * 16 * ND + f)])
                    g = codes >> 3
                    idx_v[pl.ds(f * R + h * H + i * 16, 16)] = g + f * GPF
                    off_v[pl.ds(f * R + h * H + i * 16, 16)] = (
                        (codes - (g << 3)) << 4
                    )

            def cont_body(i, _):
                e = i * 16 + iota
                r = e // NCONT
                j = e - NCONT * r
                src = NF + e + (ND - NCONT) * r
                vals = plsc.load_gather(x_v, [src]).astype(jnp.float32)
                plsc.store_scatter(out_v, [h * H + r, EMB + j], vals)
                return 0

            lax.fori_loop(0, H * NCONT // 16, cont_body, 0)

        cps = [
            pltpu.async_copy(
                w_hbm.at[idx_v.at[pl.ds(f * R, R)]], gbufs[f % 2], sem_g
            )
            for f in range(2)
        ]

        for f in range(NF):
            cps[f % 2].wait()

            def ext_body(g, _, f=f):
                o16 = off_v[pl.ds(f * R + g * 16, 16)]
                for k in range(16):
                    r = g * 16 + k
                    out_v[r, pl.ds(f * E, 16)] = gbufs[f % 2][
                        r, pl.ds(o16[k], 16)
                    ]
                return 0

            lax.fori_loop(0, R // 16, ext_body, 0)

            if f + 2 < NF:
                cps[f % 2] = pltpu.async_copy(
                    w_hbm.at[idx_v.at[pl.ds((f + 2) * R, R)]],
                    gbufs[f % 2],
                    sem_g,
                )

        pltpu.sync_copy(out_v, out_hbm.at[pl.ds(base, R)])
        return 0

    lax.fori_loop(0, NCHUNK, chunk_body, 0)


def kernel(x, W):
    xf = x.reshape(-1)
    wt = W.transpose(0, 2, 1)
    w128 = _tc_relayout(wt).reshape(NF * GPF, 128)
    return _sc_embed(xf, w128)


# speedup vs baseline: 1.3483x; 1.3483x over previous
import functools

import jax
import jax.numpy as jnp
from jax import lax
from jax.experimental import pallas as pl
from jax.experimental.pallas import tpu as pltpu
from jax.experimental.pallas import tpu_sc as plsc

B = 16384
NF = 26
VOCAB = 100000
E = 16
ND = 100
NCONT = ND - NF
OUT = NF * E + NCONT
EMB = NF * E
Q = 104

NC = 2
NS = 16
NW = NC * NS
BPW = B // NW
R = 128
NCHUNK = BPW // R


@functools.partial(
    pl.kernel,
    mesh=plsc.VectorSubcoreMesh(core_axis_name="c", subcore_axis_name="s"),
    out_type=jax.ShapeDtypeStruct((B, OUT), jnp.float32),
    compiler_params=pltpu.CompilerParams(
        use_tc_tiling_on_sc=False, needs_layout_passes=False
    ),
    scratch_types=[
        pltpu.VMEM((R // 2 * ND,), jnp.int32),
        pltpu.VMEM((R * EMB,), jnp.int32),
        pltpu.VMEM((R, OUT), jnp.float32),
        pltpu.SemaphoreType.DMA,
        pltpu.SemaphoreType.DMA,
    ],
)
def _sc_embed(x_hbm, w_hbm, out_hbm, x_v, idx_v, out_v, sem_in, sem_g):
    wid = lax.axis_index("s") * NC + lax.axis_index("c")
    iota = lax.iota(jnp.int32, 16)
    evec = iota * VOCAB

    H = R // 2

    def chunk_body(c, _):
        base = wid * BPW + c * R

        for h in range(2):
            pltpu.async_copy(
                x_hbm.at[pl.ds((base + h * H) * ND, H * ND)], x_v, sem_in
            ).wait()

            def idx_body(r, _):
                ro = (h * H + r) * EMB
                row = r * ND
                for f in range(NF):
                    v16 = x_v[pl.ds(row + f, 16)]
                    idx_v[pl.ds(ro + f * E, 16)] = evec + (
                        v16[0] + f * VOCAB * E
                    )
                return 0

            lax.fori_loop(0, H, idx_body, 0)

            def cont_body(i, _):
                e = i * 16 + iota
                r = e // NCONT
                j = e - NCONT * r
                src = NF + e + (ND - NCONT) * r
                vals = plsc.load_gather(x_v, [src]).astype(jnp.float32)
                plsc.store_scatter(out_v, [h * H + r, EMB + j], vals)
                return 0

            lax.fori_loop(0, H * NCONT // 16, cont_body, 0)

        def fire_body(r, _):
            for q in range(EMB // Q):
                pltpu.async_copy(
                    w_hbm.at[idx_v.at[pl.ds(r * EMB + q * Q, Q)]],
                    out_v.at[r, pl.ds(q * Q, Q)],
                    sem_g,
                )
            return 0

        lax.fori_loop(0, R, fire_body, 0)

        pltpu.make_async_copy(
            x_hbm.at[pl.ds(0, R * EMB)], idx_v, sem_g
        ).wait()

        pltpu.sync_copy(out_v, out_hbm.at[pl.ds(base, R)])
        return 0

    lax.fori_loop(0, NCHUNK, chunk_body, 0)


def kernel(x, W):
    xf = x.reshape(-1)
    wt = W.transpose(0, 2, 1).reshape(-1)
    return _sc_embed(xf, wt)
